# Initial kernel scaffold; baseline (speedup 1.0000x reference)
import jax
import jax.numpy as jnp
from jax.experimental import pallas as pl


def kernel(triples, nodes):
    raise NotImplementedError("write your pallas kernel here")


# speedup vs baseline: 1.1340x; 1.1340x over previous
import jax
import jax.numpy as jnp
from jax import lax
from jax.experimental import pallas as pl
from jax.experimental.pallas import tpu as pltpu
from jax.experimental.pallas import tpu_sc as plsc

_N_TRIPLES = 320000
_D = 128
_NC = 2
_NS = 16
_NW = _NC * _NS
_C = 128
_NCHUNKS = _N_TRIPLES // _C
_CHUNKS_PER_W = -(-_NCHUNKS // _NW)
_G = _C // 16


def _body(s_idx_hbm, o_idx_hbm, nodes_hbm, out_hbm,
          sidx_v, oidx_v, srows_v, orows_v, out_v, sem_s, sem_o):
    cid = lax.axis_index("c")
    sid = lax.axis_index("s")
    wid = sid * _NC + cid

    lanes = lax.iota(jnp.int32, 16)

    def chunk(c, carry):
        cn = c * _NW + wid

        @pl.when(cn < _NCHUNKS)
        def _():
            cbase = cn * _C
            pltpu.sync_copy(s_idx_hbm.at[pl.ds(cbase, _C)], sidx_v)
            pltpu.sync_copy(o_idx_hbm.at[pl.ds(cbase, _C)], oidx_v)
            cps = pltpu.async_copy(nodes_hbm.at[sidx_v], srows_v, sem_s)
            cpo = pltpu.async_copy(nodes_hbm.at[oidx_v], orows_v, sem_o)
            cps.wait()
            cpo.wait()

            def group(g, gcarry):
                rvec = g * 16 + lanes

                def dstep(dd, acc):
                    for u in range(8):
                        dvec = jnp.full((16,), dd * 8 + u, dtype=jnp.int32)
                        sv = plsc.load_gather(srows_v, [rvec, dvec])
                        ov = plsc.load_gather(orows_v, [rvec, dvec])
                        acc = acc + sv * ov
                    return acc

                acc = lax.fori_loop(0, _D // 8, dstep,
                                    jnp.zeros((16,), jnp.float32))
                out_v[pl.ds(g * 16, 16)] = acc
                return gcarry

            lax.fori_loop(0, _G, group, 0)
            pltpu.sync_copy(out_v, out_hbm.at[pl.ds(cbase, _C)])

        return carry

    lax.fori_loop(0, _CHUNKS_PER_W, chunk, 0)


def kernel(triples, nodes):
    s_idx = triples[:, 0]
    o_idx = triples[:, 2]
    mesh = plsc.VectorSubcoreMesh(core_axis_name="c", subcore_axis_name="s")
    f = pl.kernel(
        _body,
        mesh=mesh,
        out_type=jax.ShapeDtypeStruct((_N_TRIPLES,), jnp.float32),
        scratch_types=[
            pltpu.VMEM((_C,), jnp.int32),
            pltpu.VMEM((_C,), jnp.int32),
            pltpu.VMEM((_C, _D), jnp.float32),
            pltpu.VMEM((_C, _D), jnp.float32),
            pltpu.VMEM((_C,), jnp.float32),
            pltpu.SemaphoreType.DMA,
            pltpu.SemaphoreType.DMA,
        ],
        compiler_params=pltpu.CompilerParams(needs_layout_passes=False),
    )
    return f(s_idx, o_idx, nodes)


# speedup vs baseline: 4.0351x; 3.5581x over previous
import jax
import jax.numpy as jnp
from jax import lax
from jax.experimental import pallas as pl
from jax.experimental.pallas import tpu as pltpu
from jax.experimental.pallas import tpu_sc as plsc

_N_TRIPLES = 320000
_D = 128
_NC = 2
_NS = 16
_NW = _NC * _NS
_C = 128
_NCHUNKS = _N_TRIPLES // _C
_CHUNKS_PER_W = -(-_NCHUNKS // _NW)
_G = _C // 16


def _body(s_idx_hbm, o_idx_hbm, nodes_hbm, out_hbm,
          sidx_v, oidx_v, srows_v, orows_v, out_v, part_v, sem_s, sem_o):
    cid = lax.axis_index("c")
    sid = lax.axis_index("s")
    wid = sid * _NC + cid

    lanes = lax.iota(jnp.int32, 16)
    lanes17 = lanes * 17

    def chunk(c, carry):
        cn = c * _NW + wid

        @pl.when(cn < _NCHUNKS)
        def _():
            cbase = cn * _C
            pltpu.sync_copy(s_idx_hbm.at[pl.ds(cbase, _C)], sidx_v)
            pltpu.sync_copy(o_idx_hbm.at[pl.ds(cbase, _C)], oidx_v)
            cps = pltpu.async_copy(nodes_hbm.at[sidx_v], srows_v, sem_s)
            cpo = pltpu.async_copy(nodes_hbm.at[oidx_v], orows_v, sem_o)
            cps.wait()
            cpo.wait()

            def group(g, gcarry):
                for i in range(16):
                    t = g * 16 + i
                    pvec = (srows_v[t, pl.ds(0, 16)] *
                            orows_v[t, pl.ds(0, 16)])
                    for j in range(1, 8):
                        pvec = pvec + (srows_v[t, pl.ds(j * 16, 16)] *
                                       orows_v[t, pl.ds(j * 16, 16)])
                    part_v[pl.ds(i * 17, 16)] = pvec
                acc = plsc.load_gather(part_v, [lanes17])
                for l in range(1, 16):
                    acc = acc + plsc.load_gather(part_v, [lanes17 + l])
                out_v[pl.ds(g * 16, 16)] = acc
                return gcarry

            lax.fori_loop(0, _G, group, 0)
            pltpu.sync_copy(out_v, out_hbm.at[pl.ds(cbase, _C)])

        return carry

    lax.fori_loop(0, _CHUNKS_PER_W, chunk, 0)


def kernel(triples, nodes):
    s_idx = triples[:, 0]
    o_idx = triples[:, 2]
    mesh = plsc.VectorSubcoreMesh(core_axis_name="c", subcore_axis_name="s")
    f = pl.kernel(
        _body,
        mesh=mesh,
        out_type=jax.ShapeDtypeStruct((_N_TRIPLES,), jnp.float32),
        scratch_types=[
            pltpu.VMEM((_C,), jnp.int32),
            pltpu.VMEM((_C,), jnp.int32),
            pltpu.VMEM((_C, _D), jnp.float32),
            pltpu.VMEM((_C, _D), jnp.float32),
            pltpu.VMEM((_C,), jnp.float32),
            pltpu.VMEM((16 * 17,), jnp.float32),
            pltpu.SemaphoreType.DMA,
            pltpu.SemaphoreType.DMA,
        ],
        compiler_params=pltpu.CompilerParams(needs_layout_passes=False),
    )
    return f(s_idx, o_idx, nodes)


# speedup vs baseline: 8.4856x; 2.1030x over previous
import jax
import jax.numpy as jnp
from jax import lax
from jax.experimental import pallas as pl
from jax.experimental.pallas import tpu as pltpu
from jax.experimental.pallas import tpu_sc as plsc

_N_TRIPLES = 320000
_D = 128
_NC = 2
_NS = 16
_NW = _NC * _NS
_PER_W = _N_TRIPLES // _NW
_C = 128
_NFULL = _PER_W // _C
_TAIL = _PER_W - _NFULL * _C
_TAIL_BASE = _NFULL * _C


def _compute_chunk(srows, orows, part_v, out_v, out_base, lanes17, ngroups):
    def group(g, gcarry):
        for i in range(16):
            t = g * 16 + i
            pvec = (srows[t, pl.ds(0, 16)] * orows[t, pl.ds(0, 16)])
            for j in range(1, 8):
                pvec = pvec + (srows[t, pl.ds(j * 16, 16)] *
                               orows[t, pl.ds(j * 16, 16)])
            part_v[pl.ds(i * 17, 16)] = pvec
        acc = plsc.load_gather(part_v, [lanes17])
        for l in range(1, 16):
            acc = acc + plsc.load_gather(part_v, [lanes17 + l])
        out_v[pl.ds(out_base + g * 16, 16)] = acc
        return gcarry

    lax.fori_loop(0, ngroups, group, 0)


def _body(s_idx_hbm, o_idx_hbm, nodes_hbm, out_hbm,
          sidx_v, oidx_v, srows0, orows0, srows1, orows1, out_v, part_v,
          sem_s0, sem_o0, sem_s1, sem_o1):
    cid = lax.axis_index("c")
    sid = lax.axis_index("s")
    wid = sid * _NC + cid
    base = wid * _PER_W

    lanes = lax.iota(jnp.int32, 16)
    lanes17 = lanes * 17

    pltpu.sync_copy(s_idx_hbm.at[pl.ds(base, _PER_W)], sidx_v)
    pltpu.sync_copy(o_idx_hbm.at[pl.ds(base, _PER_W)], oidx_v)

    def start_gathers(chunk, srows, orows, sem_s, sem_o):
        pltpu.async_copy(nodes_hbm.at[sidx_v.at[pl.ds(chunk * _C, _C)]],
                         srows, sem_s)
        pltpu.async_copy(nodes_hbm.at[oidx_v.at[pl.ds(chunk * _C, _C)]],
                         orows, sem_o)

    def wait_gathers(srows, orows, sem_s, sem_o):
        pltpu.make_async_copy(nodes_hbm.at[pl.ds(0, _C)], srows, sem_s).wait()
        pltpu.make_async_copy(nodes_hbm.at[pl.ds(0, _C)], orows, sem_o).wait()

    start_gathers(0, srows0, orows0, sem_s0, sem_o0)

    def outer(gg, carry):
        g0 = 2 * gg
        start_gathers(g0 + 1, srows1, orows1, sem_s1, sem_o1)
        wait_gathers(srows0, orows0, sem_s0, sem_o0)
        _compute_chunk(srows0, orows0, part_v, out_v, g0 * _C, lanes17, 8)

        @pl.when(gg < _NFULL // 2 - 1)
        def _():
            start_gathers(g0 + 2, srows0, orows0, sem_s0, sem_o0)

        wait_gathers(srows1, orows1, sem_s1, sem_o1)
        _compute_chunk(srows1, orows1, part_v, out_v, (g0 + 1) * _C,
                       lanes17, 8)
        return carry

    lax.fori_loop(0, _NFULL // 2, outer, 0)

    pltpu.async_copy(nodes_hbm.at[sidx_v.at[pl.ds(_TAIL_BASE, _TAIL)]],
                     srows0.at[pl.ds(0, _TAIL)], sem_s0)
    pltpu.async_copy(nodes_hbm.at[oidx_v.at[pl.ds(_TAIL_BASE, _TAIL)]],
                     orows0.at[pl.ds(0, _TAIL)], sem_o0)
    pltpu.make_async_copy(nodes_hbm.at[pl.ds(0, _TAIL)],
                          srows0.at[pl.ds(0, _TAIL)], sem_s0).wait()
    pltpu.make_async_copy(nodes_hbm.at[pl.ds(0, _TAIL)],
                          orows0.at[pl.ds(0, _TAIL)], sem_o0).wait()
    _compute_chunk(srows0, orows0, part_v, out_v, _TAIL_BASE, lanes17, 1)

    pltpu.sync_copy(out_v, out_hbm.at[pl.ds(base, _PER_W)])


def kernel(triples, nodes):
    s_idx = triples[:, 0]
    o_idx = triples[:, 2]
    mesh = plsc.VectorSubcoreMesh(core_axis_name="c", subcore_axis_name="s")
    f = pl.kernel(
        _body,
        mesh=mesh,
        out_type=jax.ShapeDtypeStruct((_N_TRIPLES,), jnp.float32),
        scratch_types=[
            pltpu.VMEM((_PER_W,), jnp.int32),
            pltpu.VMEM((_PER_W,), jnp.int32),
            pltpu.VMEM((_C, _D), jnp.float32),
            pltpu.VMEM((_C, _D), jnp.float32),
            pltpu.VMEM((_C, _D), jnp.float32),
            pltpu.VMEM((_C, _D), jnp.float32),
            pltpu.VMEM((_PER_W,), jnp.float32),
            pltpu.VMEM((16 * 17,), jnp.float32),
            pltpu.SemaphoreType.DMA,
            pltpu.SemaphoreType.DMA,
            pltpu.SemaphoreType.DMA,
            pltpu.SemaphoreType.DMA,
        ],
        compiler_params=pltpu.CompilerParams(needs_layout_passes=False),
    )
    return f(s_idx, o_idx, nodes)
